# baseline (speedup 1.0000x reference)
import functools

import jax
import jax.numpy as jnp
from jax import lax
from jax.experimental import pallas as pl
from jax.experimental.pallas import tpu as pltpu
from jax.experimental.pallas import tpu_sc as plsc

NROWS = 64
NCOLS = 1_000_000
LANES = 16
TILE_R = 8
TILE_C = 128
SHARD_TILES = 1080
SHARD_COLS = SHARD_TILES * TILE_C
SC_COLS = 4 * SHARD_COLS
T = 9
NCH = SHARD_TILES // T
CHUNK_COLS = T * TILE_C
NSLOT = 4
NGRP = NCH // NSLOT
TC_BLOCK = 2048
TC_BLK0 = SC_COLS // TC_BLOCK
TC_NB = -(-(NCOLS - SC_COLS) // TC_BLOCK)
NEG_INF = float("-inf")
IMAX = 2**31 - 1

_mesh = plsc.VectorSubcoreMesh(core_axis_name="c", subcore_axis_name="s")


@functools.partial(
    pl.kernel,
    mesh=_mesh,
    out_type=(jax.ShapeDtypeStruct((4 * TILE_C,), jnp.float32),
              jax.ShapeDtypeStruct((4 * TILE_C,), jnp.int32)),
    scratch_types=[
        pltpu.VMEM((TILE_R, CHUNK_COLS), jnp.float32),
        pltpu.VMEM((TILE_R, CHUNK_COLS), jnp.float32),
        pltpu.VMEM((TILE_R, CHUNK_COLS), jnp.float32),
        pltpu.VMEM((TILE_R, CHUNK_COLS), jnp.float32),
        pltpu.VMEM((TILE_R, CHUNK_COLS), jnp.float32),
        pltpu.VMEM((TILE_R, CHUNK_COLS), jnp.float32),
        pltpu.VMEM((TILE_R, CHUNK_COLS), jnp.float32),
        pltpu.VMEM((TILE_R, CHUNK_COLS), jnp.float32),
        pltpu.VMEM((LANES,), jnp.float32),
        pltpu.VMEM((LANES,), jnp.int32),
        pltpu.SemaphoreType.DMA,
        pltpu.SemaphoreType.DMA,
        pltpu.SemaphoreType.DMA,
        pltpu.SemaphoreType.DMA,
        pltpu.SemaphoreType.DMA,
        pltpu.SemaphoreType.DMA,
        pltpu.SemaphoreType.DMA,
        pltpu.SemaphoreType.DMA,
    ],
)
def _gumbel_argmax_sc(scores_hbm, gumbel_hbm, outv_hbm, outi_hbm,
                      s0, s1, s2, s3, g0, g1, g2, g3, stage_v, stage_i,
                      sem_s0, sem_s1, sem_s2, sem_s3,
                      sem_g0, sem_g1, sem_g2, sem_g3):
    core = lax.axis_index("c")
    sub = lax.axis_index("s")
    band = core * 4 + sub // 4
    q = sub % 4
    row0 = band * TILE_R
    shard0 = q * SHARD_COLS

    sbufs = (s0, s1, s2, s3)
    gbufs = (g0, g1, g2, g3)
    ssems = (sem_s0, sem_s1, sem_s2, sem_s3)
    gsems = (sem_g0, sem_g1, sem_g2, sem_g3)

    def start(chunk, slot):
        c0 = shard0 + chunk * CHUNK_COLS
        pltpu.async_copy(
            scores_hbm.at[pl.ds(row0, TILE_R), pl.ds(c0, CHUNK_COLS)],
            sbufs[slot], ssems[slot])
        pltpu.async_copy(
            gumbel_hbm.at[pl.ds(row0, TILE_R), pl.ds(c0, CHUNK_COLS)],
            gbufs[slot], gsems[slot])

    def wait(slot):
        pltpu.make_async_copy(
            scores_hbm.at[pl.ds(0, TILE_R), pl.ds(0, CHUNK_COLS)],
            sbufs[slot], ssems[slot]).wait()
        pltpu.make_async_copy(
            gumbel_hbm.at[pl.ds(0, TILE_R), pl.ds(0, CHUNK_COLS)],
            gbufs[slot], gsems[slot]).wait()

    idx0 = lax.iota(jnp.int32, LANES)

    def compute(slot, chunk, carry):
        sb = sbufs[slot]
        gb = gbufs[slot]
        cbase = shard0 + chunk * CHUNK_COLS
        ms, bis = carry
        ms = list(ms)
        bis = list(bis)

        for r in range(TILE_R):
            def rbody(t, rc, r=r):
                m, bi = rc
                tbase = cbase + t * TILE_C
                for c in range(TILE_C // LANES):
                    o = t * TILE_C + c * LANES
                    p = sb[r, pl.ds(o, LANES)] + gb[r, pl.ds(o, LANES)]
                    upd = p > m
                    iv = idx0 + (tbase + c * LANES)
                    m = jnp.where(upd, p, m)
                    bi = jnp.where(upd, iv, bi)
                return m, bi

            ms[r], bis[r] = lax.fori_loop(0, T, rbody, (ms[r], bis[r]))
        return tuple(ms), tuple(bis)

    m_init = tuple(jnp.full((LANES,), NEG_INF, jnp.float32)
                   for _ in range(TILE_R))
    b_init = tuple(jnp.zeros((LANES,), jnp.int32) for _ in range(TILE_R))

    start(0, 0)
    start(1, 1)
    start(2, 2)

    def grp_body(p, carry):
        for j in range(NSLOT):
            idx = NSLOT * p + j
            wait(j)
            carry = compute(j, idx, carry)

            @pl.when(idx + NSLOT - 1 < NCH)
            def _(idx=idx, j=j):
                start(idx + NSLOT - 1, (j + NSLOT - 1) % NSLOT)
        return carry

    ms, bis = lax.fori_loop(0, NGRP, grp_body, (m_init, b_init))
    ms = list(ms)
    bis = list(bis)

    for r in range(TILE_R):
        m, bi = ms[r], bis[r]
        for shift in (1, 2, 4, 8):
            perm = idx0 ^ shift
            om = m.at[perm].get(mode="promise_in_bounds")
            obi = bi.at[perm].get(mode="promise_in_bounds")
            upd = (om > m) | ((om == m) & (obi < bi))
            m = jnp.where(upd, om, m)
            bi = jnp.where(upd, obi, bi)
        ms[r] = m
        bis[r] = bi

    valv = jnp.full((LANES,), NEG_INF, jnp.float32)
    idxv = jnp.zeros((LANES,), jnp.int32)
    for r in range(TILE_R):
        lane_r = idx0 == r
        valv = jnp.where(lane_r, ms[r], valv)
        idxv = jnp.where(lane_r, bis[r], idxv)

    stage_v[...] = valv
    stage_i[...] = idxv
    off = q * TILE_C + band * LANES
    pltpu.sync_copy(stage_v, outv_hbm.at[pl.ds(off, LANES)])
    pltpu.sync_copy(stage_i, outi_hbm.at[pl.ds(off, LANES)])


def _tc_scan_body(s_ref, g_ref, ov_ref, oi_ref, m_ref, bi_ref):
    i = pl.program_id(0)

    @pl.when(i == 0)
    def _():
        m_ref[...] = jnp.full((NROWS, TILE_C), NEG_INF, jnp.float32)
        bi_ref[...] = jnp.zeros((NROWS, TILE_C), jnp.int32)

    p = s_ref[...] + g_ref[...]
    col = (jax.lax.broadcasted_iota(jnp.int32, (NROWS, TC_BLOCK), 1)
           + (i + TC_BLK0) * TC_BLOCK)
    p = jnp.where(col < NCOLS, p, NEG_INF)
    m = m_ref[...]
    bi = bi_ref[...]
    for sub in range(TC_BLOCK // TILE_C):
        ps = p[:, sub * TILE_C:(sub + 1) * TILE_C]
        cs = col[:, sub * TILE_C:(sub + 1) * TILE_C]
        upd = ps > m
        m = jnp.where(upd, ps, m)
        bi = jnp.where(upd, cs, bi)
    m_ref[...] = m
    bi_ref[...] = bi

    @pl.when(i == TC_NB - 1)
    def _():
        rowmax = jnp.max(m, axis=1, keepdims=True)
        cand = jnp.where(m == rowmax, bi, IMAX)
        ov_ref[...] = rowmax
        oi_ref[...] = jnp.min(cand, axis=1, keepdims=True)


_tc_scan = pl.pallas_call(
    _tc_scan_body,
    grid=(TC_NB,),
    in_specs=[pl.BlockSpec((NROWS, TC_BLOCK), lambda i: (0, i + TC_BLK0)),
              pl.BlockSpec((NROWS, TC_BLOCK), lambda i: (0, i + TC_BLK0))],
    out_specs=[pl.BlockSpec((NROWS, 1), lambda i: (0, 0)),
               pl.BlockSpec((NROWS, 1), lambda i: (0, 0))],
    out_shape=[jax.ShapeDtypeStruct((NROWS, 1), jnp.float32),
               jax.ShapeDtypeStruct((NROWS, 1), jnp.int32)],
    scratch_shapes=[pltpu.VMEM((NROWS, TILE_C), jnp.float32),
                    pltpu.VMEM((NROWS, TILE_C), jnp.int32)],
)


def _merge4_body(v_ref, i_ref, ov_ref, oi_ref):
    bv = v_ref[0:1, :]
    bi = i_ref[0:1, :]
    for j in range(1, 4):
        v = v_ref[j:j + 1, :]
        ii = i_ref[j:j + 1, :]
        upd = v > bv
        bv = jnp.where(upd, v, bv)
        bi = jnp.where(upd, ii, bi)
    ov_ref[...] = bv
    oi_ref[...] = bi


_merge4_tc = pl.pallas_call(
    _merge4_body,
    out_shape=[jax.ShapeDtypeStruct((1, TILE_C), jnp.float32),
               jax.ShapeDtypeStruct((1, TILE_C), jnp.int32)],
)


def _merge2_body(sv_ref, si_ref, tv_ref, ti_ref, o_ref):
    upd = tv_ref[...] > sv_ref[...]
    o_ref[...] = jnp.where(upd, ti_ref[...], si_ref[...])


_merge2_tc = pl.pallas_call(
    _merge2_body,
    out_shape=jax.ShapeDtypeStruct((NROWS, 1), jnp.int32),
)


def kernel(scores, gumbel):
    tcv, tci = _tc_scan(scores, gumbel)
    scv, sci = _gumbel_argmax_sc(scores, gumbel)
    mv, mi = _merge4_tc(scv.reshape(4, TILE_C), sci.reshape(4, TILE_C))
    mv64 = mv.reshape(TILE_R, LANES)[:, :TILE_R].reshape(NROWS, 1)
    mi64 = mi.reshape(TILE_R, LANES)[:, :TILE_R].reshape(NROWS, 1)
    return _merge2_tc(mv64, mi64, tcv, tci)


# speedup vs baseline: 1.0555x; 1.0555x over previous
import functools

import jax
import jax.numpy as jnp
from jax import lax
from jax.experimental import pallas as pl
from jax.experimental.pallas import tpu as pltpu
from jax.experimental.pallas import tpu_sc as plsc

NROWS = 64
NCOLS = 1_000_000
LANES = 16
TILE_R = 8
TILE_C = 128
FULL_TILES = NCOLS // TILE_C
SHARD_TILES = FULL_TILES // 4
SHARD_COLS = SHARD_TILES * TILE_C
TAIL_COL0 = FULL_TILES * TILE_C
TAIL_W = NCOLS - TAIL_COL0
T = 9
NCH = SHARD_TILES // T
CHUNK_COLS = T * TILE_C
NSLOT = 6
NGRP = NCH // NSLOT
NEG_INF = float("-inf")

_mesh = plsc.VectorSubcoreMesh(core_axis_name="c", subcore_axis_name="s")


@functools.partial(
    pl.kernel,
    mesh=_mesh,
    out_type=(jax.ShapeDtypeStruct((4 * TILE_C,), jnp.float32),
              jax.ShapeDtypeStruct((4 * TILE_C,), jnp.int32)),
    scratch_types=[
        pltpu.VMEM((TILE_R, CHUNK_COLS), jnp.float32),
        pltpu.VMEM((TILE_R, CHUNK_COLS), jnp.float32),
        pltpu.VMEM((TILE_R, CHUNK_COLS), jnp.float32),
        pltpu.VMEM((TILE_R, CHUNK_COLS), jnp.float32),
        pltpu.VMEM((TILE_R, CHUNK_COLS), jnp.float32),
        pltpu.VMEM((TILE_R, CHUNK_COLS), jnp.float32),
        pltpu.VMEM((TILE_R, CHUNK_COLS), jnp.float32),
        pltpu.VMEM((TILE_R, CHUNK_COLS), jnp.float32),
        pltpu.VMEM((TILE_R, CHUNK_COLS), jnp.float32),
        pltpu.VMEM((TILE_R, CHUNK_COLS), jnp.float32),
        pltpu.VMEM((TILE_R, CHUNK_COLS), jnp.float32),
        pltpu.VMEM((TILE_R, CHUNK_COLS), jnp.float32),
        pltpu.VMEM((TILE_R, TILE_C), jnp.float32),
        pltpu.VMEM((TILE_R, TILE_C), jnp.float32),
        pltpu.VMEM((LANES,), jnp.float32),
        pltpu.VMEM((LANES,), jnp.int32),
        pltpu.SemaphoreType.DMA,
        pltpu.SemaphoreType.DMA,
        pltpu.SemaphoreType.DMA,
        pltpu.SemaphoreType.DMA,
        pltpu.SemaphoreType.DMA,
        pltpu.SemaphoreType.DMA,
        pltpu.SemaphoreType.DMA,
        pltpu.SemaphoreType.DMA,
        pltpu.SemaphoreType.DMA,
        pltpu.SemaphoreType.DMA,
        pltpu.SemaphoreType.DMA,
        pltpu.SemaphoreType.DMA,
        pltpu.SemaphoreType.DMA,
        pltpu.SemaphoreType.DMA,
    ],
)
def _gumbel_argmax(scores_hbm, gumbel_hbm, stail_hbm, gtail_hbm,
                   outv_hbm, outi_hbm,
                   s0, s1, s2, s3, s4, s5, g0, g1, g2, g3, g4, g5,
                   ts, tg, stage_v, stage_i,
                   sem_s0, sem_s1, sem_s2, sem_s3, sem_s4, sem_s5,
                   sem_g0, sem_g1, sem_g2, sem_g3, sem_g4, sem_g5,
                   sem_ts, sem_tg):
    core = lax.axis_index("c")
    sub = lax.axis_index("s")
    band = core * 4 + sub // 4
    q = sub % 4
    row0 = band * TILE_R
    shard0 = q * SHARD_COLS

    sbufs = (s0, s1, s2, s3, s4, s5)
    gbufs = (g0, g1, g2, g3, g4, g5)
    ssems = (sem_s0, sem_s1, sem_s2, sem_s3, sem_s4, sem_s5)
    gsems = (sem_g0, sem_g1, sem_g2, sem_g3, sem_g4, sem_g5)

    def start(chunk, slot):
        c0 = shard0 + chunk * CHUNK_COLS
        pltpu.async_copy(
            scores_hbm.at[pl.ds(row0, TILE_R), pl.ds(c0, CHUNK_COLS)],
            sbufs[slot], ssems[slot])
        pltpu.async_copy(
            gumbel_hbm.at[pl.ds(row0, TILE_R), pl.ds(c0, CHUNK_COLS)],
            gbufs[slot], gsems[slot])

    def wait(slot):
        pltpu.make_async_copy(
            scores_hbm.at[pl.ds(0, TILE_R), pl.ds(0, CHUNK_COLS)],
            sbufs[slot], ssems[slot]).wait()
        pltpu.make_async_copy(
            gumbel_hbm.at[pl.ds(0, TILE_R), pl.ds(0, CHUNK_COLS)],
            gbufs[slot], gsems[slot]).wait()

    idx0 = lax.iota(jnp.int32, LANES)

    def compute(slot, chunk, carry):
        sb = sbufs[slot]
        gb = gbufs[slot]
        cbase = shard0 + chunk * CHUNK_COLS
        ms, bis = carry
        ms = list(ms)
        bis = list(bis)

        for r in range(TILE_R):
            def rbody(t, rc, r=r):
                m, bi = rc
                tbase = cbase + t * TILE_C
                for c in range(TILE_C // LANES):
                    o = t * TILE_C + c * LANES
                    p = sb[r, pl.ds(o, LANES)] + gb[r, pl.ds(o, LANES)]
                    upd = p > m
                    iv = idx0 + (tbase + c * LANES)
                    m = jnp.where(upd, p, m)
                    bi = jnp.where(upd, iv, bi)
                return m, bi

            ms[r], bis[r] = lax.fori_loop(0, T, rbody, (ms[r], bis[r]))
        return tuple(ms), tuple(bis)

    m_init = tuple(jnp.full((LANES,), NEG_INF, jnp.float32)
                   for _ in range(TILE_R))
    b_init = tuple(jnp.zeros((LANES,), jnp.int32) for _ in range(TILE_R))

    pltpu.async_copy(stail_hbm.at[pl.ds(row0, TILE_R), :], ts, sem_ts)
    pltpu.async_copy(gtail_hbm.at[pl.ds(row0, TILE_R), :], tg, sem_tg)

    for k in range(NSLOT - 1):
        start(k, k)

    def grp_body(p, carry):
        for j in range(NSLOT):
            idx = NSLOT * p + j
            wait(j)
            carry = compute(j, idx, carry)

            @pl.when(idx + NSLOT - 1 < NCH)
            def _(idx=idx, j=j):
                start(idx + NSLOT - 1, (j + NSLOT - 1) % NSLOT)
        return carry

    ms, bis = lax.fori_loop(0, NGRP, grp_body, (m_init, b_init))
    ms = list(ms)
    bis = list(bis)
    wait((NCH - 1) % NSLOT)
    (ms, bis) = [list(x) for x in compute((NCH - 1) % NSLOT, NCH - 1,
                                          (tuple(ms), tuple(bis)))]

    pltpu.make_async_copy(
        stail_hbm.at[pl.ds(0, TILE_R), :], ts, sem_ts).wait()
    pltpu.make_async_copy(
        gtail_hbm.at[pl.ds(0, TILE_R), :], tg, sem_tg).wait()
    for r in range(TILE_R):
        for c in range(TILE_C // LANES):
            p = ts[r, pl.ds(c * LANES, LANES)] + tg[r, pl.ds(c * LANES, LANES)]
            upd = p > ms[r]
            iv = idx0 + (TAIL_COL0 + c * LANES)
            ms[r] = jnp.where(upd, p, ms[r])
            bis[r] = jnp.where(upd, iv, bis[r])

    for r in range(TILE_R):
        m, bi = ms[r], bis[r]
        for shift in (1, 2, 4, 8):
            perm = idx0 ^ shift
            om = m.at[perm].get(mode="promise_in_bounds")
            obi = bi.at[perm].get(mode="promise_in_bounds")
            upd = (om > m) | ((om == m) & (obi < bi))
            m = jnp.where(upd, om, m)
            bi = jnp.where(upd, obi, bi)
        ms[r] = m
        bis[r] = bi

    valv = jnp.full((LANES,), NEG_INF, jnp.float32)
    idxv = jnp.zeros((LANES,), jnp.int32)
    for r in range(TILE_R):
        lane_r = idx0 == r
        valv = jnp.where(lane_r, ms[r], valv)
        idxv = jnp.where(lane_r, bis[r], idxv)

    stage_v[...] = valv
    stage_i[...] = idxv
    off = q * TILE_C + band * LANES
    pltpu.sync_copy(stage_v, outv_hbm.at[pl.ds(off, LANES)])
    pltpu.sync_copy(stage_i, outi_hbm.at[pl.ds(off, LANES)])


def _merge_body(v_ref, i_ref, o_ref):
    bv = v_ref[0:1, :]
    bi = i_ref[0:1, :]
    for j in range(1, 4):
        v = v_ref[j:j + 1, :]
        ii = i_ref[j:j + 1, :]
        upd = v > bv
        bv = jnp.where(upd, v, bv)
        bi = jnp.where(upd, ii, bi)
    o_ref[...] = bi


_merge_tc = pl.pallas_call(
    _merge_body,
    out_shape=jax.ShapeDtypeStruct((1, TILE_C), jnp.int32),
)


def kernel(scores, gumbel):
    stail = jnp.concatenate(
        [scores[:, TAIL_COL0:],
         jnp.full((NROWS, TILE_C - TAIL_W), NEG_INF, jnp.float32)], axis=1)
    gtail = jnp.concatenate(
        [gumbel[:, TAIL_COL0:],
         jnp.zeros((NROWS, TILE_C - TAIL_W), jnp.float32)], axis=1)
    outv, outi = _gumbel_argmax(scores, gumbel, stail, gtail)
    merged = _merge_tc(outv.reshape(4, TILE_C), outi.reshape(4, TILE_C))
    return merged.reshape(TILE_R, LANES)[:, :TILE_R].reshape(NROWS, 1)
